# baseline (speedup 1.0000x reference)
import functools

import jax
import jax.numpy as jnp
from jax import lax
from jax.experimental import pallas as pl
from jax.experimental.pallas import tpu as pltpu
from jax.experimental.pallas import tpu_sc as plsc

NC = 2
NS = 16
C = 80


def _segment_sum_sc(table, packed3, n_rows):
    _, nchunks, _ = packed3.shape
    d = table.shape[1]
    n_pad = ((n_rows + NS * 8 - 1) // (NS * 8)) * (NS * 8)
    rows_per_tile = n_pad // NS
    zr = 8
    mesh = plsc.VectorSubcoreMesh(core_axis_name="c", subcore_axis_name="s")

    @functools.partial(
        pl.kernel,
        out_type=jax.ShapeDtypeStruct((NC, n_pad, d), jnp.float32),
        mesh=mesh,
        compiler_params=pltpu.CompilerParams(use_tc_tiling_on_sc=False),
        scratch_types=[
            pltpu.VMEM((nchunks, C), jnp.int32),
            pltpu.VMEM((C,), jnp.int32),
            pltpu.VMEM((C,), jnp.int32),
            pltpu.VMEM((C,), jnp.int32),
            pltpu.VMEM((C,), jnp.int32),
            pltpu.VMEM((C, d), jnp.float32),
            pltpu.VMEM((C, d), jnp.float32),
            pltpu.VMEM((zr, d), jnp.float32),
            pltpu.VMEM_SHARED((n_pad, d), jnp.float32),
            pltpu.SemaphoreType.DMA,
            pltpu.SemaphoreType.DMA,
            pltpu.SemaphoreType.DMA,
            pltpu.SemaphoreType.DMA,
            pltpu.SemaphoreType.DMA,
        ],
    )
    def seg_sum(table_hbm, idx_hbm, out_hbm,
                idx_v, srcb0, srcb1, dstb0, dstb1, rows0, rows1, zbuf, acc,
                gsem0, gsem1, ssem0, ssem1, zsem):
        cid = lax.axis_index("c")
        sid = lax.axis_index("s")
        tile = cid * NS + sid

        pltpu.sync_copy(idx_hbm.at[tile], idx_v)

        zero16 = jnp.zeros((16,), jnp.float32)

        def zrow(r, carry):
            for j in range(d // 16):
                zbuf[r, pl.ds(j * 16, 16)] = zero16
            return carry

        lax.fori_loop(0, zr, zrow, 0)
        row0 = sid * rows_per_tile
        nz = rows_per_tile // zr

        def zfire(k, carry):
            pltpu.async_copy(zbuf, acc.at[pl.ds(row0 + k * zr, zr)], zsem)
            return carry

        lax.fori_loop(0, nz, zfire, 0)

        def zdrain(k, carry):
            pltpu.make_async_copy(zbuf, acc.at[pl.ds(row0, zr)], zsem).wait()
            return carry

        lax.fori_loop(0, nz, zdrain, 0)
        plsc.subcore_barrier()

        def gstart(i, srcb, dstb, buf, gsem):
            for k in range(C // 16):
                t = idx_v[i, pl.ds(k * 16, 16)]
                srcb[pl.ds(k * 16, 16)] = lax.shift_right_logical(t, 16)
                dstb[pl.ds(k * 16, 16)] = lax.bitwise_and(t, 0xFFFF)
            pltpu.async_copy(table_hbm.at[srcb], buf, gsem)

        def gwait_sstart(srcb, dstb, buf, gsem, ssem):
            pltpu.make_async_copy(table_hbm.at[srcb], buf, gsem).wait()
            pltpu.async_copy(buf, acc.at[dstb], ssem, add=True)

        def swait(dstb, buf, ssem):
            pltpu.make_async_copy(buf, acc.at[dstb], ssem).wait()

        gstart(0, srcb0, dstb0, rows0, gsem0)
        gstart(1, srcb1, dstb1, rows1, gsem1)

        def body(j, carry):
            i = 2 * j
            gwait_sstart(srcb0, dstb0, rows0, gsem0, ssem0)
            gwait_sstart(srcb1, dstb1, rows1, gsem1, ssem1)
            swait(dstb0, rows0, ssem0)
            gstart(i + 2, srcb0, dstb0, rows0, gsem0)
            swait(dstb1, rows1, ssem1)
            gstart(i + 3, srcb1, dstb1, rows1, gsem1)
            return carry

        lax.fori_loop(0, (nchunks - 3) // 2, body, 0)
        gwait_sstart(srcb0, dstb0, rows0, gsem0, ssem0)
        gwait_sstart(srcb1, dstb1, rows1, gsem1, ssem1)
        swait(dstb0, rows0, ssem0)
        gstart(nchunks - 1, srcb0, dstb0, rows0, gsem0)
        gwait_sstart(srcb0, dstb0, rows0, gsem0, ssem0)
        swait(dstb1, rows1, ssem1)
        swait(dstb0, rows0, ssem0)
        plsc.subcore_barrier()

        pltpu.sync_copy(acc.at[pl.ds(row0, rows_per_tile)],
                        out_hbm.at[cid, pl.ds(row0, rows_per_tile)])

    return seg_sum(table, packed3)


def _mlp0_tc(x, p, eps, Wa, ba, Wb, bb, block_rows=2000):
    n, din = x.shape
    h = Wa.shape[1]

    def body(eps_ref, x_ref, p_ref, wa_ref, ba_ref, wb_ref, bb_ref, o_ref):
        t = (1.0 + eps_ref[0, 0]) * x_ref[...] + p_ref[0] + p_ref[1]
        t = jnp.dot(t, wa_ref[...], preferred_element_type=jnp.float32) + ba_ref[...]
        t = jnp.maximum(t, 0.0)
        t = jnp.dot(t, wb_ref[...], preferred_element_type=jnp.float32) + bb_ref[...]
        o_ref[...] = jnp.maximum(t, 0.0)

    grid = (n // block_rows,)
    return pl.pallas_call(
        body,
        grid=grid,
        in_specs=[
            pl.BlockSpec(memory_space=pltpu.SMEM),
            pl.BlockSpec((block_rows, din), lambda i: (i, 0)),
            pl.BlockSpec((2, block_rows, din), lambda i: (0, i, 0)),
            pl.BlockSpec(Wa.shape, lambda i: (0, 0)),
            pl.BlockSpec(ba.shape, lambda i: (0, 0)),
            pl.BlockSpec(Wb.shape, lambda i: (0, 0)),
            pl.BlockSpec(bb.shape, lambda i: (0, 0)),
        ],
        out_specs=pl.BlockSpec((block_rows, h), lambda i: (i, 0)),
        out_shape=jax.ShapeDtypeStruct((n, h), jnp.float32),
    )(eps, x, p, Wa, ba, Wb, bb)


def _mlp1_tc(hin, q, eps, Wa, ba, Wb, bb, block_rows=2000):
    n, h = hin.shape
    dout = Wb.shape[1]

    def body(eps_ref, h_ref, q_ref, wa_ref, ba_ref, wb_ref, bb_ref, o_ref):
        t = (1.0 + eps_ref[0, 0]) * h_ref[...] + q_ref[0] + q_ref[1]
        t = jnp.dot(t, wa_ref[...], preferred_element_type=jnp.float32) + ba_ref[...]
        t = jnp.maximum(t, 0.0)
        z = jnp.dot(t, wb_ref[...], preferred_element_type=jnp.float32) + bb_ref[...]
        m = jnp.max(z, axis=1, keepdims=True)
        e = z - m
        o_ref[...] = e - jnp.log(jnp.sum(jnp.exp(e), axis=1, keepdims=True))

    grid = (n // block_rows,)
    return pl.pallas_call(
        body,
        grid=grid,
        in_specs=[
            pl.BlockSpec(memory_space=pltpu.SMEM),
            pl.BlockSpec((block_rows, h), lambda i: (i, 0)),
            pl.BlockSpec((2, block_rows, h), lambda i: (0, i, 0)),
            pl.BlockSpec(Wa.shape, lambda i: (0, 0)),
            pl.BlockSpec(ba.shape, lambda i: (0, 0)),
            pl.BlockSpec(Wb.shape, lambda i: (0, 0)),
            pl.BlockSpec(bb.shape, lambda i: (0, 0)),
        ],
        out_specs=pl.BlockSpec((block_rows, dout), lambda i: (i, 0)),
        out_shape=jax.ShapeDtypeStruct((n, dout), jnp.float32),
    )(eps, hin, q, Wa, ba, Wb, bb)


def kernel(x, edge_index, eps0, eps1, W0a, b0a, W0b, b0b, W1a, b1a, W1b, b1b):
    n = x.shape[0]
    e = edge_index.shape[1]
    packed3 = (edge_index[0] * 65536 + edge_index[1]).reshape(
        NC * NS, e // (NC * NS * C), C)
    eps0_s = eps0.reshape(1, 1)
    eps1_s = eps1.reshape(1, 1)

    p = _segment_sum_sc(x, packed3, n)
    h = _mlp0_tc(x, p, eps0_s, W0a, b0a.reshape(1, -1),
                 W0b, b0b.reshape(1, -1))
    q = _segment_sum_sc(h, packed3, n)
    return _mlp1_tc(h, q, eps1_s, W1a, b1a.reshape(1, -1),
                    W1b, b1b.reshape(1, -1))


# speedup vs baseline: 1.1924x; 1.1924x over previous
import functools

import jax
import jax.numpy as jnp
from jax import lax
from jax.experimental import pallas as pl
from jax.experimental.pallas import tpu as pltpu
from jax.experimental.pallas import tpu_sc as plsc

NC = 2
NS = 16
C = 80


def _segment_sum_sc(table, packed3, n_rows):
    _, nchunks, _ = packed3.shape
    d = table.shape[1]
    n_pad = ((n_rows + NS * 8 - 1) // (NS * 8)) * (NS * 8)
    rows_per_tile = n_pad // NS
    zr = 8
    mesh = plsc.VectorSubcoreMesh(core_axis_name="c", subcore_axis_name="s")

    @functools.partial(
        pl.kernel,
        out_type=jax.ShapeDtypeStruct((NC, n_pad, d), jnp.float32),
        mesh=mesh,
        compiler_params=pltpu.CompilerParams(use_tc_tiling_on_sc=False),
        scratch_types=[
            pltpu.VMEM((nchunks, C), jnp.int32),
            pltpu.VMEM((C,), jnp.int32),
            pltpu.VMEM((C,), jnp.int32),
            pltpu.VMEM((C,), jnp.int32),
            pltpu.VMEM((C,), jnp.int32),
            pltpu.VMEM((C, d), jnp.float32),
            pltpu.VMEM((C, d), jnp.float32),
            pltpu.VMEM((zr, d), jnp.float32),
            pltpu.VMEM_SHARED((n_pad, d), jnp.float32),
            pltpu.SemaphoreType.DMA,
            pltpu.SemaphoreType.DMA,
            pltpu.SemaphoreType.DMA,
            pltpu.SemaphoreType.DMA,
            pltpu.SemaphoreType.DMA,
        ],
    )
    def seg_sum(table_hbm, idx_hbm, out_hbm,
                idx_v, srcb0, srcb1, dstb0, dstb1, rows0, rows1, zbuf, acc,
                gsem0, gsem1, ssem0, ssem1, zsem):
        cid = lax.axis_index("c")
        sid = lax.axis_index("s")
        tile = cid * NS + sid

        pltpu.sync_copy(idx_hbm.at[tile], idx_v)

        zero16 = jnp.zeros((16,), jnp.float32)

        def zrow(r, carry):
            for j in range(d // 16):
                zbuf[r, pl.ds(j * 16, 16)] = zero16
            return carry

        lax.fori_loop(0, zr, zrow, 0)
        row0 = sid * rows_per_tile
        nz = rows_per_tile // zr

        def zfire(k, carry):
            pltpu.async_copy(zbuf, acc.at[pl.ds(row0 + k * zr, zr)], zsem)
            return carry

        lax.fori_loop(0, nz, zfire, 0)

        def zdrain(k, carry):
            pltpu.make_async_copy(zbuf, acc.at[pl.ds(row0, zr)], zsem).wait()
            return carry

        lax.fori_loop(0, nz, zdrain, 0)
        plsc.subcore_barrier()

        def gstart(i, srcb, dstb, buf, gsem):
            for k in range(C // 16):
                t = idx_v[i, pl.ds(k * 16, 16)]
                srcb[pl.ds(k * 16, 16)] = lax.shift_right_logical(t, 16)
                dstb[pl.ds(k * 16, 16)] = lax.bitwise_and(t, 0xFFFF)
            pltpu.async_copy(table_hbm.at[srcb], buf, gsem)

        def finish(srcb, dstb, buf, gsem):
            pltpu.make_async_copy(table_hbm.at[srcb], buf, gsem).wait()
            pltpu.sync_copy(buf, acc.at[dstb], add=True)

        gstart(0, srcb0, dstb0, rows0, gsem0)
        gstart(1, srcb1, dstb1, rows1, gsem1)

        def body(j, carry):
            i = 2 * j
            finish(srcb0, dstb0, rows0, gsem0)
            gstart(i + 2, srcb0, dstb0, rows0, gsem0)
            finish(srcb1, dstb1, rows1, gsem1)
            gstart(i + 3, srcb1, dstb1, rows1, gsem1)
            return carry

        lax.fori_loop(0, (nchunks - 3) // 2, body, 0)
        finish(srcb0, dstb0, rows0, gsem0)
        gstart(nchunks - 1, srcb0, dstb0, rows0, gsem0)
        finish(srcb1, dstb1, rows1, gsem1)
        finish(srcb0, dstb0, rows0, gsem0)
        plsc.subcore_barrier()

        pltpu.sync_copy(acc.at[pl.ds(row0, rows_per_tile)],
                        out_hbm.at[cid, pl.ds(row0, rows_per_tile)])

    return seg_sum(table, packed3)


def _mlp0_tc(x, p, eps, Wa, ba, Wb, bb, block_rows=2000):
    n, din = x.shape
    h = Wa.shape[1]

    def body(eps_ref, x_ref, p_ref, wa_ref, ba_ref, wb_ref, bb_ref, o_ref):
        t = (1.0 + eps_ref[0, 0]) * x_ref[...] + p_ref[0] + p_ref[1]
        t = jnp.dot(t, wa_ref[...], preferred_element_type=jnp.float32) + ba_ref[...]
        t = jnp.maximum(t, 0.0)
        t = jnp.dot(t, wb_ref[...], preferred_element_type=jnp.float32) + bb_ref[...]
        o_ref[...] = jnp.maximum(t, 0.0)

    grid = (n // block_rows,)
    return pl.pallas_call(
        body,
        grid=grid,
        in_specs=[
            pl.BlockSpec(memory_space=pltpu.SMEM),
            pl.BlockSpec((block_rows, din), lambda i: (i, 0)),
            pl.BlockSpec((2, block_rows, din), lambda i: (0, i, 0)),
            pl.BlockSpec(Wa.shape, lambda i: (0, 0)),
            pl.BlockSpec(ba.shape, lambda i: (0, 0)),
            pl.BlockSpec(Wb.shape, lambda i: (0, 0)),
            pl.BlockSpec(bb.shape, lambda i: (0, 0)),
        ],
        out_specs=pl.BlockSpec((block_rows, h), lambda i: (i, 0)),
        out_shape=jax.ShapeDtypeStruct((n, h), jnp.float32),
    )(eps, x, p, Wa, ba, Wb, bb)


def _mlp1_tc(hin, q, eps, Wa, ba, Wb, bb, block_rows=2000):
    n, h = hin.shape
    dout = Wb.shape[1]

    def body(eps_ref, h_ref, q_ref, wa_ref, ba_ref, wb_ref, bb_ref, o_ref):
        t = (1.0 + eps_ref[0, 0]) * h_ref[...] + q_ref[0] + q_ref[1]
        t = jnp.dot(t, wa_ref[...], preferred_element_type=jnp.float32) + ba_ref[...]
        t = jnp.maximum(t, 0.0)
        z = jnp.dot(t, wb_ref[...], preferred_element_type=jnp.float32) + bb_ref[...]
        m = jnp.max(z, axis=1, keepdims=True)
        e = z - m
        o_ref[...] = e - jnp.log(jnp.sum(jnp.exp(e), axis=1, keepdims=True))

    grid = (n // block_rows,)
    return pl.pallas_call(
        body,
        grid=grid,
        in_specs=[
            pl.BlockSpec(memory_space=pltpu.SMEM),
            pl.BlockSpec((block_rows, h), lambda i: (i, 0)),
            pl.BlockSpec((2, block_rows, h), lambda i: (0, i, 0)),
            pl.BlockSpec(Wa.shape, lambda i: (0, 0)),
            pl.BlockSpec(ba.shape, lambda i: (0, 0)),
            pl.BlockSpec(Wb.shape, lambda i: (0, 0)),
            pl.BlockSpec(bb.shape, lambda i: (0, 0)),
        ],
        out_specs=pl.BlockSpec((block_rows, dout), lambda i: (i, 0)),
        out_shape=jax.ShapeDtypeStruct((n, dout), jnp.float32),
    )(eps, hin, q, Wa, ba, Wb, bb)


def kernel(x, edge_index, eps0, eps1, W0a, b0a, W0b, b0b, W1a, b1a, W1b, b1b):
    n = x.shape[0]
    e = edge_index.shape[1]
    packed3 = (edge_index[0] * 65536 + edge_index[1]).reshape(
        NC * NS, e // (NC * NS * C), C)
    eps0_s = eps0.reshape(1, 1)
    eps1_s = eps1.reshape(1, 1)

    p = _segment_sum_sc(x, packed3, n)
    h = _mlp0_tc(x, p, eps0_s, W0a, b0a.reshape(1, -1),
                 W0b, b0b.reshape(1, -1))
    q = _segment_sum_sc(h, packed3, n)
    return _mlp1_tc(h, q, eps1_s, W1a, b1a.reshape(1, -1),
                    W1b, b1b.reshape(1, -1))


# speedup vs baseline: 1.2033x; 1.0092x over previous
import functools

import jax
import jax.numpy as jnp
from jax import lax
from jax.experimental import pallas as pl
from jax.experimental.pallas import tpu as pltpu
from jax.experimental.pallas import tpu_sc as plsc

NC = 2
NS = 16
C = 80


def _segment_sum_sc(table, packed3, n_rows, stage_table=False):
    _, nchunks, _ = packed3.shape
    d = table.shape[1]
    n_pad = ((n_rows + NS * 8 - 1) // (NS * 8)) * (NS * 8)
    rows_per_tile = n_pad // NS
    zr = 8
    mesh = plsc.VectorSubcoreMesh(core_axis_name="c", subcore_axis_name="s")

    scratch = [
        pltpu.VMEM((nchunks, C), jnp.int32),
        pltpu.VMEM((C,), jnp.int32),
        pltpu.VMEM((C,), jnp.int32),
        pltpu.VMEM((C,), jnp.int32),
        pltpu.VMEM((C,), jnp.int32),
        pltpu.VMEM((C, d), jnp.float32),
        pltpu.VMEM((C, d), jnp.float32),
        pltpu.VMEM((zr, d), jnp.float32),
        pltpu.VMEM_SHARED((n_pad, d), jnp.float32),
        pltpu.SemaphoreType.DMA,
        pltpu.SemaphoreType.DMA,
        pltpu.SemaphoreType.DMA,
        pltpu.SemaphoreType.DMA,
        pltpu.SemaphoreType.DMA,
    ]
    if stage_table:
        scratch.append(pltpu.VMEM_SHARED((n_rows, d), jnp.float32))

    @functools.partial(
        pl.kernel,
        out_type=jax.ShapeDtypeStruct((NC, n_pad, d), jnp.float32),
        mesh=mesh,
        compiler_params=pltpu.CompilerParams(use_tc_tiling_on_sc=False),
        scratch_types=scratch,
    )
    def seg_sum(table_hbm, idx_hbm, out_hbm,
                idx_v, srcb0, srcb1, dstb0, dstb1, rows0, rows1, zbuf, acc,
                gsem0, gsem1, ssem0, ssem1, zsem, *maybe_ts):
        cid = lax.axis_index("c")
        sid = lax.axis_index("s")
        tile = cid * NS + sid

        pltpu.sync_copy(idx_hbm.at[tile], idx_v)

        if stage_table:
            table_src = maybe_ts[0]
            spt = n_rows // NS
            pltpu.sync_copy(table_hbm.at[pl.ds(sid * spt, spt)],
                            table_src.at[pl.ds(sid * spt, spt)])
        else:
            table_src = table_hbm

        zero16 = jnp.zeros((16,), jnp.float32)

        def zrow(r, carry):
            for j in range(d // 16):
                zbuf[r, pl.ds(j * 16, 16)] = zero16
            return carry

        lax.fori_loop(0, zr, zrow, 0)
        row0 = sid * rows_per_tile
        nz = rows_per_tile // zr

        def zfire(k, carry):
            pltpu.async_copy(zbuf, acc.at[pl.ds(row0 + k * zr, zr)], zsem)
            return carry

        lax.fori_loop(0, nz, zfire, 0)

        def zdrain(k, carry):
            pltpu.make_async_copy(zbuf, acc.at[pl.ds(row0, zr)], zsem).wait()
            return carry

        lax.fori_loop(0, nz, zdrain, 0)
        plsc.subcore_barrier()

        def gstart(i, srcb, dstb, buf, gsem):
            for k in range(C // 16):
                t = idx_v[i, pl.ds(k * 16, 16)]
                srcb[pl.ds(k * 16, 16)] = lax.shift_right_logical(t, 16)
                dstb[pl.ds(k * 16, 16)] = lax.bitwise_and(t, 0xFFFF)
            pltpu.async_copy(table_src.at[srcb], buf, gsem)

        def finish(srcb, dstb, buf, gsem):
            pltpu.make_async_copy(table_src.at[srcb], buf, gsem).wait()
            pltpu.sync_copy(buf, acc.at[dstb], add=True)

        gstart(0, srcb0, dstb0, rows0, gsem0)
        gstart(1, srcb1, dstb1, rows1, gsem1)

        def body(j, carry):
            i = 2 * j
            finish(srcb0, dstb0, rows0, gsem0)
            gstart(i + 2, srcb0, dstb0, rows0, gsem0)
            finish(srcb1, dstb1, rows1, gsem1)
            gstart(i + 3, srcb1, dstb1, rows1, gsem1)
            return carry

        lax.fori_loop(0, (nchunks - 3) // 2, body, 0)
        finish(srcb0, dstb0, rows0, gsem0)
        gstart(nchunks - 1, srcb0, dstb0, rows0, gsem0)
        finish(srcb1, dstb1, rows1, gsem1)
        finish(srcb0, dstb0, rows0, gsem0)
        plsc.subcore_barrier()

        pltpu.sync_copy(acc.at[pl.ds(row0, rows_per_tile)],
                        out_hbm.at[cid, pl.ds(row0, rows_per_tile)])

    return seg_sum(table, packed3)


def _mlp0_tc(x, p, eps, Wa, ba, Wb, bb, block_rows=2000):
    n, din = x.shape
    h = Wa.shape[1]

    def body(eps_ref, x_ref, p_ref, wa_ref, ba_ref, wb_ref, bb_ref, o_ref):
        t = (1.0 + eps_ref[0, 0]) * x_ref[...] + p_ref[0] + p_ref[1]
        t = jnp.dot(t, wa_ref[...], preferred_element_type=jnp.float32) + ba_ref[...]
        t = jnp.maximum(t, 0.0)
        t = jnp.dot(t, wb_ref[...], preferred_element_type=jnp.float32) + bb_ref[...]
        o_ref[...] = jnp.maximum(t, 0.0)

    grid = (n // block_rows,)
    return pl.pallas_call(
        body,
        grid=grid,
        in_specs=[
            pl.BlockSpec(memory_space=pltpu.SMEM),
            pl.BlockSpec((block_rows, din), lambda i: (i, 0)),
            pl.BlockSpec((2, block_rows, din), lambda i: (0, i, 0)),
            pl.BlockSpec(Wa.shape, lambda i: (0, 0)),
            pl.BlockSpec(ba.shape, lambda i: (0, 0)),
            pl.BlockSpec(Wb.shape, lambda i: (0, 0)),
            pl.BlockSpec(bb.shape, lambda i: (0, 0)),
        ],
        out_specs=pl.BlockSpec((block_rows, h), lambda i: (i, 0)),
        out_shape=jax.ShapeDtypeStruct((n, h), jnp.float32),
    )(eps, x, p, Wa, ba, Wb, bb)


def _mlp1_tc(hin, q, eps, Wa, ba, Wb, bb, block_rows=2000):
    n, h = hin.shape
    dout = Wb.shape[1]

    def body(eps_ref, h_ref, q_ref, wa_ref, ba_ref, wb_ref, bb_ref, o_ref):
        t = (1.0 + eps_ref[0, 0]) * h_ref[...] + q_ref[0] + q_ref[1]
        t = jnp.dot(t, wa_ref[...], preferred_element_type=jnp.float32) + ba_ref[...]
        t = jnp.maximum(t, 0.0)
        z = jnp.dot(t, wb_ref[...], preferred_element_type=jnp.float32) + bb_ref[...]
        m = jnp.max(z, axis=1, keepdims=True)
        e = z - m
        o_ref[...] = e - jnp.log(jnp.sum(jnp.exp(e), axis=1, keepdims=True))

    grid = (n // block_rows,)
    return pl.pallas_call(
        body,
        grid=grid,
        in_specs=[
            pl.BlockSpec(memory_space=pltpu.SMEM),
            pl.BlockSpec((block_rows, h), lambda i: (i, 0)),
            pl.BlockSpec((2, block_rows, h), lambda i: (0, i, 0)),
            pl.BlockSpec(Wa.shape, lambda i: (0, 0)),
            pl.BlockSpec(ba.shape, lambda i: (0, 0)),
            pl.BlockSpec(Wb.shape, lambda i: (0, 0)),
            pl.BlockSpec(bb.shape, lambda i: (0, 0)),
        ],
        out_specs=pl.BlockSpec((block_rows, dout), lambda i: (i, 0)),
        out_shape=jax.ShapeDtypeStruct((n, dout), jnp.float32),
    )(eps, hin, q, Wa, ba, Wb, bb)


def kernel(x, edge_index, eps0, eps1, W0a, b0a, W0b, b0b, W1a, b1a, W1b, b1b):
    n = x.shape[0]
    e = edge_index.shape[1]
    packed3 = (edge_index[0] * 65536 + edge_index[1]).reshape(
        NC * NS, e // (NC * NS * C), C)
    eps0_s = eps0.reshape(1, 1)
    eps1_s = eps1.reshape(1, 1)

    p = _segment_sum_sc(x, packed3, n)
    h = _mlp0_tc(x, p, eps0_s, W0a, b0a.reshape(1, -1),
                 W0b, b0b.reshape(1, -1))
    q = _segment_sum_sc(h, packed3, n, stage_table=True)
    return _mlp1_tc(h, q, eps1_s, W1a, b1a.reshape(1, -1),
                    W1b, b1b.reshape(1, -1))


# speedup vs baseline: 1.2085x; 1.0043x over previous
import functools

import jax
import jax.numpy as jnp
from jax import lax
from jax.experimental import pallas as pl
from jax.experimental.pallas import tpu as pltpu
from jax.experimental.pallas import tpu_sc as plsc

NC = 2
NS = 16
C = 80


def _segment_sum_sc(table, packed3, n_rows, stage_table=False, nslots=2):
    _, nchunks, _ = packed3.shape
    d = table.shape[1]
    n_pad = ((n_rows + NS * 8 - 1) // (NS * 8)) * (NS * 8)
    rows_per_tile = n_pad // NS
    zr = 8
    mesh = plsc.VectorSubcoreMesh(core_axis_name="c", subcore_axis_name="s")

    scratch = (
        [pltpu.VMEM((nchunks, C), jnp.int32)]
        + [pltpu.VMEM((C,), jnp.int32)] * nslots
        + [pltpu.VMEM((C,), jnp.int32)] * nslots
        + [pltpu.VMEM((C, d), jnp.float32)] * nslots
        + [
            pltpu.VMEM((zr, d), jnp.float32),
            pltpu.VMEM_SHARED((n_pad, d), jnp.float32),
        ]
        + [pltpu.SemaphoreType.DMA] * nslots
        + [pltpu.SemaphoreType.DMA]
    )
    if stage_table:
        scratch.append(pltpu.VMEM_SHARED((n_rows, d), jnp.float32))

    @functools.partial(
        pl.kernel,
        out_type=jax.ShapeDtypeStruct((NC, n_pad, d), jnp.float32),
        mesh=mesh,
        compiler_params=pltpu.CompilerParams(use_tc_tiling_on_sc=False),
        scratch_types=scratch,
    )
    def seg_sum(table_hbm, idx_hbm, out_hbm, idx_v, *rest):
        srcbs = rest[0:nslots]
        dstbs = rest[nslots:2 * nslots]
        rowss = rest[2 * nslots:3 * nslots]
        zbuf = rest[3 * nslots]
        acc = rest[3 * nslots + 1]
        gsems = rest[3 * nslots + 2:4 * nslots + 2]
        zsem = rest[4 * nslots + 2]
        maybe_ts = rest[4 * nslots + 3:]
        cid = lax.axis_index("c")
        sid = lax.axis_index("s")
        tile = cid * NS + sid

        pltpu.sync_copy(idx_hbm.at[tile], idx_v)

        if stage_table:
            table_src = maybe_ts[0]
            spt = n_rows // NS
            pltpu.sync_copy(table_hbm.at[pl.ds(sid * spt, spt)],
                            table_src.at[pl.ds(sid * spt, spt)])
        else:
            table_src = table_hbm

        zero16 = jnp.zeros((16,), jnp.float32)

        def zrow(r, carry):
            for j in range(d // 16):
                zbuf[r, pl.ds(j * 16, 16)] = zero16
            return carry

        lax.fori_loop(0, zr, zrow, 0)
        row0 = sid * rows_per_tile
        nz = rows_per_tile // zr

        def zfire(k, carry):
            pltpu.async_copy(zbuf, acc.at[pl.ds(row0 + k * zr, zr)], zsem)
            return carry

        lax.fori_loop(0, nz, zfire, 0)

        def zdrain(k, carry):
            pltpu.make_async_copy(zbuf, acc.at[pl.ds(row0, zr)], zsem).wait()
            return carry

        lax.fori_loop(0, nz, zdrain, 0)
        plsc.subcore_barrier()

        def gstart(i, s):
            for k in range(C // 16):
                t = idx_v[i, pl.ds(k * 16, 16)]
                srcbs[s][pl.ds(k * 16, 16)] = lax.shift_right_logical(t, 16)
                dstbs[s][pl.ds(k * 16, 16)] = lax.bitwise_and(t, 0xFFFF)
            pltpu.async_copy(table_src.at[srcbs[s]], rowss[s], gsems[s])

        def finish(s):
            pltpu.make_async_copy(table_src.at[srcbs[s]], rowss[s], gsems[s]).wait()
            pltpu.sync_copy(rowss[s], acc.at[dstbs[s]], add=True)

        for s in range(nslots):
            gstart(s, s)
        loop_iters = (nchunks - 2 * nslots) // nslots

        def body(j, carry):
            i = j * nslots
            for s in range(nslots):
                finish(s)
                gstart(i + s + nslots, s)
            return carry

        lax.fori_loop(0, loop_iters, body, 0)
        for i in range(loop_iters * nslots, nchunks):
            s = i % nslots
            finish(s)
            if i + nslots < nchunks:
                gstart(i + nslots, s)
        plsc.subcore_barrier()

        pltpu.sync_copy(acc.at[pl.ds(row0, rows_per_tile)],
                        out_hbm.at[cid, pl.ds(row0, rows_per_tile)])

    return seg_sum(table, packed3)


def _mlp0_tc(x, p, eps, Wa, ba, Wb, bb, block_rows=2000):
    n, din = x.shape
    h = Wa.shape[1]

    def body(eps_ref, x_ref, p_ref, wa_ref, ba_ref, wb_ref, bb_ref, o_ref):
        t = (1.0 + eps_ref[0, 0]) * x_ref[...] + p_ref[0] + p_ref[1]
        t = jnp.dot(t, wa_ref[...], preferred_element_type=jnp.float32) + ba_ref[...]
        t = jnp.maximum(t, 0.0)
        t = jnp.dot(t, wb_ref[...], preferred_element_type=jnp.float32) + bb_ref[...]
        o_ref[...] = jnp.maximum(t, 0.0)

    grid = (n // block_rows,)
    return pl.pallas_call(
        body,
        grid=grid,
        in_specs=[
            pl.BlockSpec(memory_space=pltpu.SMEM),
            pl.BlockSpec((block_rows, din), lambda i: (i, 0)),
            pl.BlockSpec((2, block_rows, din), lambda i: (0, i, 0)),
            pl.BlockSpec(Wa.shape, lambda i: (0, 0)),
            pl.BlockSpec(ba.shape, lambda i: (0, 0)),
            pl.BlockSpec(Wb.shape, lambda i: (0, 0)),
            pl.BlockSpec(bb.shape, lambda i: (0, 0)),
        ],
        out_specs=pl.BlockSpec((block_rows, h), lambda i: (i, 0)),
        out_shape=jax.ShapeDtypeStruct((n, h), jnp.float32),
    )(eps, x, p, Wa, ba, Wb, bb)


def _mlp1_tc(hin, q, eps, Wa, ba, Wb, bb, block_rows=2000):
    n, h = hin.shape
    dout = Wb.shape[1]

    def body(eps_ref, h_ref, q_ref, wa_ref, ba_ref, wb_ref, bb_ref, o_ref):
        t = (1.0 + eps_ref[0, 0]) * h_ref[...] + q_ref[0] + q_ref[1]
        t = jnp.dot(t, wa_ref[...], preferred_element_type=jnp.float32) + ba_ref[...]
        t = jnp.maximum(t, 0.0)
        z = jnp.dot(t, wb_ref[...], preferred_element_type=jnp.float32) + bb_ref[...]
        m = jnp.max(z, axis=1, keepdims=True)
        e = z - m
        o_ref[...] = e - jnp.log(jnp.sum(jnp.exp(e), axis=1, keepdims=True))

    grid = (n // block_rows,)
    return pl.pallas_call(
        body,
        grid=grid,
        in_specs=[
            pl.BlockSpec(memory_space=pltpu.SMEM),
            pl.BlockSpec((block_rows, h), lambda i: (i, 0)),
            pl.BlockSpec((2, block_rows, h), lambda i: (0, i, 0)),
            pl.BlockSpec(Wa.shape, lambda i: (0, 0)),
            pl.BlockSpec(ba.shape, lambda i: (0, 0)),
            pl.BlockSpec(Wb.shape, lambda i: (0, 0)),
            pl.BlockSpec(bb.shape, lambda i: (0, 0)),
        ],
        out_specs=pl.BlockSpec((block_rows, dout), lambda i: (i, 0)),
        out_shape=jax.ShapeDtypeStruct((n, dout), jnp.float32),
    )(eps, hin, q, Wa, ba, Wb, bb)


def kernel(x, edge_index, eps0, eps1, W0a, b0a, W0b, b0b, W1a, b1a, W1b, b1b):
    n = x.shape[0]
    e = edge_index.shape[1]
    packed3 = (edge_index[0] * 65536 + edge_index[1]).reshape(
        NC * NS, e // (NC * NS * C), C)
    eps0_s = eps0.reshape(1, 1)
    eps1_s = eps1.reshape(1, 1)

    p = _segment_sum_sc(x, packed3, n)
    h = _mlp0_tc(x, p, eps0_s, W0a, b0a.reshape(1, -1),
                 W0b, b0b.reshape(1, -1))
    q = _segment_sum_sc(h, packed3, n, stage_table=True, nslots=3)
    return _mlp1_tc(h, q, eps1_s, W1a, b1a.reshape(1, -1),
                    W1b, b1b.reshape(1, -1))
